# baseline (speedup 1.0000x reference)
import functools

import jax
from jax import lax
import jax.numpy as jnp
from jax.experimental import pallas as pl
from jax.experimental.pallas import tpu as pltpu
from jax.experimental.pallas import tpu_sc as plsc

_B, _N, _J, _K, _DC, _W = 4, 8192, 1024, 5, 69, 256
_TILE = 1024
_INT_MAX = jnp.iinfo(jnp.int32).max
_NWORKERS = 32
_CHUNK = _B * _N // 2 // _NWORKERS
_LANES = 16

_INTERPRET = False


def _tm_body(nodesT_ref, cond_ref, ro_ref, trans_ref, scale_ref,
             w0aT_ref, w0bT_ref, b0T_ref, w1T_ref, b1T_ref, w2T_ref, b2T_ref,
             out_ref, a_ref):
    f32 = jnp.float32
    nodesT = nodesT_ref[...]
    h = jnp.dot(w0aT_ref[...], nodesT, preferred_element_type=f32)
    cond_c = jnp.dot(w0bT_ref[...], cond_ref[0], preferred_element_type=f32)
    h = jnp.maximum(h + cond_c + b0T_ref[...], 0.0)
    h = jnp.maximum(jnp.dot(w1T_ref[...], h, preferred_element_type=f32)
                    + b1T_ref[...], 0.0)
    tfT = jnp.dot(w2T_ref[...], h, preferred_element_type=f32) + b2T_ref[...]

    a0, a1, a2 = tfT[0:1, :], tfT[1:2, :], tfT[2:3, :]
    t0, t1, t2 = tfT[3:4, :], tfT[4:5, :], tfT[5:6, :]

    def rodrigues(a0, a1, a2):
        norm = jnp.sqrt((a0 + 1e-8) ** 2 + (a1 + 1e-8) ** 2 + (a2 + 1e-8) ** 2)
        half = norm * 0.5
        cw = jnp.cos(half)
        sw = jnp.sin(half)
        inv = sw / norm
        qw, qx, qy, qz = cw, a0 * inv, a1 * inv, a2 * inv
        qn = jax.lax.rsqrt(qw * qw + qx * qx + qy * qy + qz * qz)
        qw, qx, qy, qz = qw * qn, qx * qn, qy * qn, qz * qn
        w2, x2, y2, z2 = qw * qw, qx * qx, qy * qy, qz * qz
        wx, wy, wz = qw * qx, qw * qy, qw * qz
        xy, xz, yz = qx * qy, qx * qz, qy * qz
        r00 = w2 + x2 - y2 - z2
        r01 = 2 * xy - 2 * wz
        r02 = 2 * wy + 2 * xz
        r10 = 2 * wz + 2 * xy
        r11 = w2 - x2 + y2 - z2
        r12 = 2 * yz - 2 * wx
        r20 = 2 * xz - 2 * wy
        r21 = 2 * wx + 2 * yz
        r22 = w2 - x2 - y2 + z2
        return ((r00, r01, r02), (r10, r11, r12), (r20, r21, r22))

    r = rodrigues(a0, a1, a2)
    g = rodrigues(ro_ref[0, 0:1, 0:1], ro_ref[0, 0:1, 1:2], ro_ref[0, 0:1, 2:3])
    s = scale_ref[0, 0:1, 0:1]

    rows = []
    for i in range(3):
        gi0, gi1, gi2 = g[i][0], g[i][1], g[i][2]
        for j in range(3):
            rows.append((gi0 * r[0][j] + gi1 * r[1][j] + gi2 * r[2][j]) * s)
        tr_i = trans_ref[0, 0:1, i:i + 1]
        rows.append((gi0 * t0 + gi1 * t1 + gi2 * t2) * s + tr_i * s)
    zero = jnp.zeros_like(a0)
    one = zero + 1.0
    rows.extend([zero, zero, zero, one])
    out_ref[0] = jnp.concatenate(rows, axis=0).T

    nn = (nodesT[0:1] * nodesT[0:1] + nodesT[1:2] * nodesT[1:2]
          + nodesT[2:3] * nodesT[2:3])
    aT = jnp.concatenate(
        [nodesT * -2.0, nn, one, zero, zero, zero], axis=0)
    a_ref[0] = aT.T


def _knn_body(xT_ref, a_ref, idx_ref, w_ref, tiles_per_batch, tile_offset):
    f32 = jnp.float32
    xT = xT_ref[...]
    x0, x1, x2 = xT[0:1, :], xT[1:2, :], xT[2:3, :]
    xx = x0 * x0 + x1 * x1 + x2 * x2
    one = jnp.ones_like(xx)
    zero = jnp.zeros_like(xx)
    R = jnp.concatenate([xT, one, xx, zero, zero, zero], axis=0)
    d2 = jnp.dot(a_ref[0], R, preferred_element_type=f32)

    ibits = jax.lax.bitcast_convert_type(d2, jnp.int32)
    iota = jax.lax.broadcasted_iota(jnp.int32, d2.shape, 0)
    key = jnp.bitwise_or(jnp.bitwise_and(ibits, -1024), iota)

    base = ((pl.program_id(0) + tile_offset) // tiles_per_batch) * _J
    kA = jnp.minimum(key[:_J // 2], key[_J // 2:])
    kB = jnp.maximum(key[:_J // 2], key[_J // 2:])
    idxs = []
    ws = []
    for k in range(_K):
        m = jnp.min(kA, axis=0, keepdims=True)
        d2k = jax.lax.bitcast_convert_type(jnp.bitwise_and(m, -1024), f32)
        dk = jnp.minimum(jnp.sqrt(jnp.maximum(d2k, 1e-12)), 1.0)
        ws.append(-jnp.log(dk - 1e-6))
        idxs.append(jnp.bitwise_and(m, 1023) + base)
        if k + 1 < _K:
            sel = kA == m
            kA = jnp.where(sel, kB, kA)
            kB = jnp.where(sel, _INT_MAX, kB)

    wmat = jnp.concatenate(ws, axis=0)
    wsum = ws[0] + ws[1] + ws[2] + ws[3] + ws[4]
    idx_ref[...] = jnp.concatenate(idxs, axis=0)
    w_ref[...] = wmat / wsum


def _sc_apply_body(table_hbm, idx_hbm, w_hbm, xT_hbm, out_hbm,
                   tab_v, idx_v, w_v, x_v, o_v):
    f32 = jnp.float32
    wid = lax.axis_index("s") * 2 + lax.axis_index("c")
    base = wid * _CHUNK
    pltpu.sync_copy(table_hbm, tab_v)
    pltpu.sync_copy(idx_hbm.at[:, pl.ds(base, _CHUNK)], idx_v)
    pltpu.sync_copy(w_hbm.at[:, pl.ds(base, _CHUNK)], w_v)
    pltpu.sync_copy(xT_hbm.at[:, pl.ds(base, _CHUNK)], x_v)

    lane3 = jax.lax.iota(jnp.int32, _LANES) * 3

    def body(g, carry):
        sl = pl.ds(g * _LANES, _LANES)
        acc = None
        for k in range(_K):
            fbase = idx_v[k, sl] * 16
            wv = w_v[k, sl]
            vals = [plsc.load_gather(tab_v, [fbase + c]) for c in range(12)]
            if acc is None:
                acc = [wv * v for v in vals]
            else:
                acc = [a + wv * v for a, v in zip(acc, vals)]
        x0 = x_v[0, sl]
        x1 = x_v[1, sl]
        x2 = x_v[2, sl]
        pos = lane3 + g * (_LANES * 3)
        plsc.store_scatter(o_v, [pos],
                           acc[0] * x0 + acc[1] * x1 + acc[2] * x2 + acc[3])
        plsc.store_scatter(o_v, [pos + 1],
                           acc[4] * x0 + acc[5] * x1 + acc[6] * x2 + acc[7])
        plsc.store_scatter(o_v, [pos + 2],
                           acc[8] * x0 + acc[9] * x1 + acc[10] * x2 + acc[11])
        return carry

    lax.fori_loop(0, _CHUNK // _LANES, body, 0)
    pltpu.sync_copy(o_v, out_hbm.at[pl.ds(base * 3, _CHUNK * 3)])


def kernel(x, nodes, cond_smpl, smpl_root_orient, smpl_trans, scale,
           W0, b0, W1, b1, W2, b2):
    B, N, _ = x.shape
    J = nodes.shape[0]
    W = W1.shape[0]
    f32 = jnp.float32

    nodesT = nodes.T
    w0aT = W0[:3].T
    w0bT = W0[3:].T
    w1T = W1.T
    w2T = jnp.pad(W2, ((0, 0), (0, 2))).T
    b0T = b0.reshape(W, 1)
    b1T = b1.reshape(W, 1)
    b2T = jnp.pad(b2, (0, 2)).reshape(8, 1)

    tm, a_mat = pl.pallas_call(
        _tm_body,
        grid=(B,),
        in_specs=[
            pl.BlockSpec((3, J), lambda b: (0, 0)),
            pl.BlockSpec((1, _DC, 1), lambda b: (b, 0, 0)),
            pl.BlockSpec((1, 1, 3), lambda b: (b, 0, 0)),
            pl.BlockSpec((1, 1, 3), lambda b: (b, 0, 0)),
            pl.BlockSpec((1, 1, 1), lambda b: (b, 0, 0)),
            pl.BlockSpec((W, 3), lambda b: (0, 0)),
            pl.BlockSpec((W, _DC), lambda b: (0, 0)),
            pl.BlockSpec((W, 1), lambda b: (0, 0)),
            pl.BlockSpec((W, W), lambda b: (0, 0)),
            pl.BlockSpec((W, 1), lambda b: (0, 0)),
            pl.BlockSpec((8, W), lambda b: (0, 0)),
            pl.BlockSpec((8, 1), lambda b: (0, 0)),
        ],
        out_specs=[
            pl.BlockSpec((1, J, 16), lambda b: (b, 0, 0)),
            pl.BlockSpec((1, J, 8), lambda b: (0, 0, 0)),
        ],
        out_shape=[
            jax.ShapeDtypeStruct((B, J, 16), f32),
            jax.ShapeDtypeStruct((1, J, 8), f32),
        ],
        interpret=_INTERPRET,
    )(nodesT, cond_smpl.reshape(B, _DC, 1), smpl_root_orient.reshape(B, 1, 3),
      smpl_trans.reshape(B, 1, 3), scale.reshape(B, 1, 1),
      w0aT, w0bT, b0T, w1T, b1T, w2T, b2T)

    BN = B * N
    half = BN // 2
    xT2d = x.reshape(BN, 3).T
    tiles_per_batch = N // _TILE
    table = tm.reshape(B * J * 16)

    mesh = plsc.VectorSubcoreMesh(core_axis_name="c", subcore_axis_name="s")
    sc_apply = functools.partial(
        pl.kernel,
        mesh=mesh,
        compiler_params=pltpu.CompilerParams(needs_layout_passes=False),
        out_type=jax.ShapeDtypeStruct((half * 3,), f32),
        scratch_types=[
            pltpu.VMEM((B * J * 16,), f32),
            pltpu.VMEM((_K, _CHUNK), jnp.int32),
            pltpu.VMEM((_K, _CHUNK), f32),
            pltpu.VMEM((3, _CHUNK), f32),
            pltpu.VMEM((_CHUNK * 3,), f32),
        ],
    )(_sc_apply_body)

    xcT_halves = []
    for h in range(2):
        xT_h = jax.lax.slice(xT2d, (0, h * half), (3, (h + 1) * half))
        idx_h, w_h = pl.pallas_call(
            functools.partial(_knn_body, tiles_per_batch=tiles_per_batch,
                              tile_offset=h * (half // _TILE)),
            grid=(half // _TILE,),
            in_specs=[
                pl.BlockSpec((3, _TILE), lambda i: (0, i)),
                pl.BlockSpec((1, J, 8), lambda i: (0, 0, 0)),
            ],
            out_specs=[
                pl.BlockSpec((_K, _TILE), lambda i: (0, i)),
                pl.BlockSpec((_K, _TILE), lambda i: (0, i)),
            ],
            out_shape=[
                jax.ShapeDtypeStruct((_K, half), jnp.int32),
                jax.ShapeDtypeStruct((_K, half), f32),
            ],
            interpret=_INTERPRET,
        )(xT_h, a_mat)
        xcT_halves.append(sc_apply(table, idx_h, w_h, xT_h))

    xc = jnp.concatenate(xcT_halves, axis=0)
    return xc.reshape(B, N, 3)


# speedup vs baseline: 1.1260x; 1.1260x over previous
import functools

import jax
from jax import lax
import jax.numpy as jnp
from jax.experimental import pallas as pl
from jax.experimental.pallas import tpu as pltpu
from jax.experimental.pallas import tpu_sc as plsc

_B, _N, _J, _K, _DC, _W = 4, 8192, 1024, 5, 69, 256
_TILE = 1024
_INT_MAX = jnp.iinfo(jnp.int32).max
_NWORKERS = 32
_CHUNK = _B * _N // 2 // _NWORKERS
_LANES = 16


def _tm_body(nodesT_ref, cond_ref, ro_ref, trans_ref, scale_ref,
             w0aT_ref, w0bT_ref, b0T_ref, w1T_ref, b1T_ref, w2T_ref, b2T_ref,
             out_ref, a_ref):
    f32 = jnp.float32
    nodesT = nodesT_ref[...]
    h = jnp.dot(w0aT_ref[...], nodesT, preferred_element_type=f32)
    cond_c = jnp.dot(w0bT_ref[...], cond_ref[0], preferred_element_type=f32)
    h = jnp.maximum(h + cond_c + b0T_ref[...], 0.0)
    h = jnp.maximum(jnp.dot(w1T_ref[...], h, preferred_element_type=f32)
                    + b1T_ref[...], 0.0)
    tfT = jnp.dot(w2T_ref[...], h, preferred_element_type=f32) + b2T_ref[...]

    a0, a1, a2 = tfT[0:1, :], tfT[1:2, :], tfT[2:3, :]
    t0, t1, t2 = tfT[3:4, :], tfT[4:5, :], tfT[5:6, :]

    def rodrigues(a0, a1, a2):
        norm = jnp.sqrt((a0 + 1e-8) ** 2 + (a1 + 1e-8) ** 2 + (a2 + 1e-8) ** 2)
        half = norm * 0.5
        cw = jnp.cos(half)
        sw = jnp.sin(half)
        inv = sw / norm
        qw, qx, qy, qz = cw, a0 * inv, a1 * inv, a2 * inv
        qn = jax.lax.rsqrt(qw * qw + qx * qx + qy * qy + qz * qz)
        qw, qx, qy, qz = qw * qn, qx * qn, qy * qn, qz * qn
        w2, x2, y2, z2 = qw * qw, qx * qx, qy * qy, qz * qz
        wx, wy, wz = qw * qx, qw * qy, qw * qz
        xy, xz, yz = qx * qy, qx * qz, qy * qz
        r00 = w2 + x2 - y2 - z2
        r01 = 2 * xy - 2 * wz
        r02 = 2 * wy + 2 * xz
        r10 = 2 * wz + 2 * xy
        r11 = w2 - x2 + y2 - z2
        r12 = 2 * yz - 2 * wx
        r20 = 2 * xz - 2 * wy
        r21 = 2 * wx + 2 * yz
        r22 = w2 - x2 - y2 + z2
        return ((r00, r01, r02), (r10, r11, r12), (r20, r21, r22))

    r = rodrigues(a0, a1, a2)
    g = rodrigues(ro_ref[0, 0:1, 0:1], ro_ref[0, 0:1, 1:2], ro_ref[0, 0:1, 2:3])
    s = scale_ref[0, 0:1, 0:1]

    rows = []
    for i in range(3):
        gi0, gi1, gi2 = g[i][0], g[i][1], g[i][2]
        for j in range(3):
            rows.append((gi0 * r[0][j] + gi1 * r[1][j] + gi2 * r[2][j]) * s)
        tr_i = trans_ref[0, 0:1, i:i + 1]
        rows.append((gi0 * t0 + gi1 * t1 + gi2 * t2) * s + tr_i * s)
    zero = jnp.zeros_like(a0)
    one = zero + 1.0
    rows.extend([zero, zero, zero, one])
    out_ref[0] = jnp.concatenate(rows, axis=0).T

    nn = (nodesT[0:1] * nodesT[0:1] + nodesT[1:2] * nodesT[1:2]
          + nodesT[2:3] * nodesT[2:3])
    aT = jnp.concatenate(
        [nodesT * -2.0, nn, one, zero, zero, zero], axis=0)
    a_ref[0] = aT.T


def _knn_body(xT_ref, a_ref, idx_ref, w_ref, tiles_per_batch, tile_offset):
    f32 = jnp.float32
    xT = xT_ref[...]
    x0, x1, x2 = xT[0:1, :], xT[1:2, :], xT[2:3, :]
    xx = x0 * x0 + x1 * x1 + x2 * x2
    one = jnp.ones_like(xx)
    zero = jnp.zeros_like(xx)
    R = jnp.concatenate([xT, one, xx, zero, zero, zero], axis=0)
    d2 = jnp.dot(a_ref[0], R, preferred_element_type=f32)

    ibits = jax.lax.bitcast_convert_type(d2, jnp.int32)
    iota = jax.lax.broadcasted_iota(jnp.int32, d2.shape, 0)
    key = jnp.bitwise_or(jnp.bitwise_and(ibits, -1024), iota)

    base = ((pl.program_id(0) + tile_offset) // tiles_per_batch) * _J
    kA = jnp.minimum(key[:_J // 2], key[_J // 2:])
    kB = jnp.maximum(key[:_J // 2], key[_J // 2:])
    idxs = []
    ws = []
    for k in range(_K):
        m = jnp.min(kA, axis=0, keepdims=True)
        d2k = jax.lax.bitcast_convert_type(jnp.bitwise_and(m, -1024), f32)
        dk = jnp.minimum(jnp.sqrt(jnp.maximum(d2k, 1e-12)), 1.0)
        ws.append(-jnp.log(dk - 1e-6))
        idxs.append(jnp.bitwise_and(m, 1023) + base)
        if k + 1 < _K:
            sel = kA == m
            kA = jnp.where(sel, kB, kA)
            kB = jnp.where(sel, _INT_MAX, kB)

    wmat = jnp.concatenate(ws, axis=0)
    wsum = ws[0] + ws[1] + ws[2] + ws[3] + ws[4]
    idx_ref[...] = jnp.concatenate(idxs, axis=0)
    w_ref[...] = wmat / wsum


def _sc_apply_body(table_hbm, idx_hbm, w_hbm, xT_hbm, out_hbm,
                   tab_v, idx_v, w_v, x_v, o_v):
    f32 = jnp.float32
    wid = lax.axis_index("s") * 2 + lax.axis_index("c")
    base = wid * _CHUNK
    pltpu.sync_copy(table_hbm, tab_v)
    pltpu.sync_copy(idx_hbm.at[:, pl.ds(base, _CHUNK)], idx_v)
    pltpu.sync_copy(w_hbm.at[:, pl.ds(base, _CHUNK)], w_v)
    pltpu.sync_copy(xT_hbm.at[:, pl.ds(base, _CHUNK)], x_v)

    def body(g, carry):
        sl = pl.ds(g * _LANES, _LANES)
        acc = None
        for k in range(_K):
            fbase = idx_v[k, sl] * 16
            wv = w_v[k, sl]
            vals = [plsc.load_gather(tab_v, [fbase + c]) for c in range(12)]
            if acc is None:
                acc = [wv * v for v in vals]
            else:
                acc = [a + wv * v for a, v in zip(acc, vals)]
        x0 = x_v[0, sl]
        x1 = x_v[1, sl]
        x2 = x_v[2, sl]
        o_v[0, sl] = acc[0] * x0 + acc[1] * x1 + acc[2] * x2 + acc[3]
        o_v[1, sl] = acc[4] * x0 + acc[5] * x1 + acc[6] * x2 + acc[7]
        o_v[2, sl] = acc[8] * x0 + acc[9] * x1 + acc[10] * x2 + acc[11]
        return carry

    lax.fori_loop(0, _CHUNK // _LANES, body, 0)
    pltpu.sync_copy(o_v, out_hbm.at[:, pl.ds(base, _CHUNK)])


def kernel(x, nodes, cond_smpl, smpl_root_orient, smpl_trans, scale,
           W0, b0, W1, b1, W2, b2):
    B, N, _ = x.shape
    J = nodes.shape[0]
    W = W1.shape[0]
    f32 = jnp.float32

    nodesT = nodes.T
    w0aT = W0[:3].T
    w0bT = W0[3:].T
    w1T = W1.T
    w2T = jnp.pad(W2, ((0, 0), (0, 2))).T
    b0T = b0.reshape(W, 1)
    b1T = b1.reshape(W, 1)
    b2T = jnp.pad(b2, (0, 2)).reshape(8, 1)

    tm, a_mat = pl.pallas_call(
        _tm_body,
        grid=(B,),
        in_specs=[
            pl.BlockSpec((3, J), lambda b: (0, 0)),
            pl.BlockSpec((1, _DC, 1), lambda b: (b, 0, 0)),
            pl.BlockSpec((1, 1, 3), lambda b: (b, 0, 0)),
            pl.BlockSpec((1, 1, 3), lambda b: (b, 0, 0)),
            pl.BlockSpec((1, 1, 1), lambda b: (b, 0, 0)),
            pl.BlockSpec((W, 3), lambda b: (0, 0)),
            pl.BlockSpec((W, _DC), lambda b: (0, 0)),
            pl.BlockSpec((W, 1), lambda b: (0, 0)),
            pl.BlockSpec((W, W), lambda b: (0, 0)),
            pl.BlockSpec((W, 1), lambda b: (0, 0)),
            pl.BlockSpec((8, W), lambda b: (0, 0)),
            pl.BlockSpec((8, 1), lambda b: (0, 0)),
        ],
        out_specs=[
            pl.BlockSpec((1, J, 16), lambda b: (b, 0, 0)),
            pl.BlockSpec((1, J, 8), lambda b: (0, 0, 0)),
        ],
        out_shape=[
            jax.ShapeDtypeStruct((B, J, 16), f32),
            jax.ShapeDtypeStruct((1, J, 8), f32),
        ],
    )(nodesT, cond_smpl.reshape(B, _DC, 1), smpl_root_orient.reshape(B, 1, 3),
      smpl_trans.reshape(B, 1, 3), scale.reshape(B, 1, 1),
      w0aT, w0bT, b0T, w1T, b1T, w2T, b2T)

    BN = B * N
    half = BN // 2
    xT2d = x.reshape(BN, 3).T
    tiles_per_batch = N // _TILE
    table = tm.reshape(B * J * 16)

    mesh = plsc.VectorSubcoreMesh(core_axis_name="c", subcore_axis_name="s")
    sc_apply = functools.partial(
        pl.kernel,
        mesh=mesh,
        compiler_params=pltpu.CompilerParams(needs_layout_passes=False),
        out_type=jax.ShapeDtypeStruct((3, half), f32),
        scratch_types=[
            pltpu.VMEM((B * J * 16,), f32),
            pltpu.VMEM((_K, _CHUNK), jnp.int32),
            pltpu.VMEM((_K, _CHUNK), f32),
            pltpu.VMEM((3, _CHUNK), f32),
            pltpu.VMEM((3, _CHUNK), f32),
        ],
    )(_sc_apply_body)

    xcT_halves = []
    for h in range(2):
        xT_h = jax.lax.slice(xT2d, (0, h * half), (3, (h + 1) * half))
        idx_h, w_h = pl.pallas_call(
            functools.partial(_knn_body, tiles_per_batch=tiles_per_batch,
                              tile_offset=h * (half // _TILE)),
            grid=(half // _TILE,),
            in_specs=[
                pl.BlockSpec((3, _TILE), lambda i: (0, i)),
                pl.BlockSpec((1, J, 8), lambda i: (0, 0, 0)),
            ],
            out_specs=[
                pl.BlockSpec((_K, _TILE), lambda i: (0, i)),
                pl.BlockSpec((_K, _TILE), lambda i: (0, i)),
            ],
            out_shape=[
                jax.ShapeDtypeStruct((_K, half), jnp.int32),
                jax.ShapeDtypeStruct((_K, half), f32),
            ],
        )(xT_h, a_mat)
        xcT_halves.append(sc_apply(table, idx_h, w_h, xT_h))

    xcT = jnp.concatenate(xcT_halves, axis=1)
    return xcT.T.reshape(B, N, 3)
